# baseline (speedup 1.0000x reference)
import functools

import jax
import jax.numpy as jnp
from jax import lax
from jax.experimental import pallas as pl
from jax.experimental.pallas import tpu as pltpu
from jax.experimental.pallas import tpu_sc as plsc

N = 100000
D = 512
NC = 2
NS = 16
NW = NC * NS
C = 120
NCHUNKS = -(-N // C)
CPW = NCHUNKS // NW
EXTRA = NCHUNKS - CPW * NW
MAXLOC = CPW + 1
IDXBUF = MAXLOC * C


def _sc_gather(x, table):
    mesh = plsc.VectorSubcoreMesh(core_axis_name="c", subcore_axis_name="s")

    @functools.partial(
        pl.kernel,
        mesh=mesh,
        out_type=jax.ShapeDtypeStruct((N, D), jnp.float32),
        scratch_types=[
            pltpu.VMEM((IDXBUF,), jnp.int32),
            pltpu.VMEM((C, D), jnp.float32),
            pltpu.VMEM((C, D), jnp.float32),
            pltpu.SemaphoreType.DMA,
            pltpu.SemaphoreType.DMA,
            pltpu.SemaphoreType.DMA,
            pltpu.SemaphoreType.DMA,
        ],
    )
    def k(x_hbm, table_hbm, out_hbm, idx_v, rows0, rows1, g0, g1, s0, s1):
        cid = lax.axis_index("c")
        sid = lax.axis_index("s")
        wid = sid * NC + cid
        nloc = CPW + jnp.where(wid < EXTRA, 1, 0)
        start = wid * CPW + jnp.minimum(wid, EXTRA)
        load_base = jnp.minimum(start * C, N - IDXBUF)

        rows = (rows0, rows1)
        gsem = (g0, g1)
        ssem = (s0, s1)

        pltpu.sync_copy(x_hbm.at[pl.ds(load_base, IDXBUF)], idx_v)

        def off_of(i):
            return jnp.minimum((start + i) * C, N - C)

        def gather(i, b):
            bo = off_of(i) - load_base
            return pltpu.make_async_copy(
                table_hbm.at[idx_v.at[pl.ds(bo, C)]], rows[b], gsem[b])

        def scatter(i, b):
            return pltpu.make_async_copy(
                rows[b], out_hbm.at[pl.ds(off_of(i), C)], ssem[b])

        gather(0, 0).start()

        @pl.when(nloc > 1)
        def _():
            gather(1, 1).start()

        def body(j, _):
            for b in range(2):
                i = 2 * j + b

                @pl.when(i < nloc)
                def _():
                    gather(i, b).wait()
                    scatter(i, b).start()

                    @pl.when(i + 2 < nloc)
                    def _():
                        scatter(i, b).wait()
                        gather(i + 2, b).start()
            return 0

        lax.fori_loop(0, (MAXLOC + 1) // 2, body, 0)

        for b in range(2):
            @pl.when(nloc > b)
            def _():
                scatter(b, b).wait()

    return k(x, table)


def kernel(x, table):
    return _sc_gather(x.astype(jnp.int32), table)
